# Initial kernel scaffold; baseline (speedup 1.0000x reference)
import jax
import jax.numpy as jnp
from jax.experimental import pallas as pl


def kernel(feature, label, centers):
    raise NotImplementedError("write your pallas kernel here")


# speedup vs baseline: 1.1844x; 1.1844x over previous
import jax
import jax.numpy as jnp
from jax.experimental import pallas as pl
from jax.experimental.pallas import tpu as pltpu

BATCH = 4096
DIM = 512
NUM_CLASSES = 10000
MARGIN = 0.35

B_TILE = 256
C_TILE = 2048
C_PAD = 10240
NB = BATCH // B_TILE
NC = C_PAD // C_TILE
NEG_BIG = -1e30


def _body(f_ref, c_ref, lbl_ref, out_ref, m_scr, s_scr, t_scr):
    j = pl.program_id(0)
    i = pl.program_id(1)

    f = f_ref[...]
    fn = f / (jnp.sqrt(jnp.sum(f * f, axis=1, keepdims=True)) + 1e-12)
    c = c_ref[...]
    cn = c / (jnp.sqrt(jnp.sum(c * c, axis=1, keepdims=True)) + 1e-12)
    logits = jax.lax.dot_general(
        fn, cn, (((1,), (1,)), ((), ())), preferred_element_type=jnp.float32
    )

    lbl = lbl_ref[0, 0, :]
    cols = j * C_TILE + jax.lax.broadcasted_iota(jnp.int32, (B_TILE, C_TILE), 1)
    is_tgt = cols == lbl[:, None]
    ml = logits - MARGIN * is_tgt.astype(jnp.float32)
    ml = jnp.where(cols < NUM_CLASSES, ml, NEG_BIG)

    tile_max = jnp.max(ml, axis=1)
    tile_t = jnp.sum(jnp.where(is_tgt, ml, 0.0), axis=1)

    @pl.when(j == 0)
    def _():
        m_scr[i] = tile_max
        s_scr[i] = jnp.sum(jnp.exp(ml - tile_max[:, None]), axis=1)
        t_scr[i] = tile_t

    @pl.when(j > 0)
    def _():
        m_old = m_scr[i]
        m_new = jnp.maximum(m_old, tile_max)
        s_scr[i] = s_scr[i] * jnp.exp(m_old - m_new) + jnp.sum(
            jnp.exp(ml - m_new[:, None]), axis=1
        )
        m_scr[i] = m_new
        t_scr[i] = t_scr[i] + tile_t

    @pl.when(j == NC - 1)
    def _():
        out_ref[0, 0, :] = m_scr[i] + jnp.log(s_scr[i]) - t_scr[i]


def kernel(feature, label, centers):
    c_pad = jnp.pad(centers, ((0, C_PAD - NUM_CLASSES), (0, 0)))
    lbl3 = label.reshape(NB, 1, B_TILE)

    out = pl.pallas_call(
        _body,
        grid=(NC, NB),
        in_specs=[
            pl.BlockSpec((B_TILE, DIM), lambda j, i: (i, 0)),
            pl.BlockSpec((C_TILE, DIM), lambda j, i: (j, 0)),
            pl.BlockSpec((1, 1, B_TILE), lambda j, i: (i, 0, 0)),
        ],
        out_specs=pl.BlockSpec((1, 1, B_TILE), lambda j, i: (i, 0, 0)),
        out_shape=jax.ShapeDtypeStruct((NB, 1, B_TILE), jnp.float32),
        scratch_shapes=[
            pltpu.VMEM((NB, B_TILE), jnp.float32),
            pltpu.VMEM((NB, B_TILE), jnp.float32),
            pltpu.VMEM((NB, B_TILE), jnp.float32),
        ],
    )(feature, c_pad, lbl3)
    return out.reshape(BATCH)


# speedup vs baseline: 1.8314x; 1.5463x over previous
import jax
import jax.numpy as jnp
from jax.experimental import pallas as pl
from jax.experimental.pallas import tpu as pltpu

BATCH = 4096
DIM = 512
NUM_CLASSES = 10000
MARGIN = 0.35

B_TILE = 512
C_TILE = 2048
C_PAD = 10240
NB = BATCH // B_TILE
NC = C_PAD // C_TILE
N_PAD = float(C_PAD - NUM_CLASSES)


def _norm_body(x_ref, o_ref):
    x = x_ref[...]
    o_ref[...] = x / (jnp.sqrt(jnp.sum(x * x, axis=1, keepdims=True)) + 1e-12)


def _row_normalize(x, row_tile):
    rows = x.shape[0]
    return pl.pallas_call(
        _norm_body,
        grid=(rows // row_tile,),
        in_specs=[pl.BlockSpec((row_tile, DIM), lambda i: (i, 0))],
        out_specs=pl.BlockSpec((row_tile, DIM), lambda i: (i, 0)),
        out_shape=jax.ShapeDtypeStruct(x.shape, jnp.float32),
    )(x)


def _body(f_ref, c_ref, lbl_ref, out_ref, s_scr, t_scr):
    j = pl.program_id(0)
    i = pl.program_id(1)

    logits = jax.lax.dot_general(
        f_ref[...], c_ref[...], (((1,), (1,)), ((), ())),
        preferred_element_type=jnp.float32,
    )

    sum_e = jnp.sum(jnp.exp(logits), axis=1)

    lbl = lbl_ref[0, 0, :]
    cols = j * C_TILE + jax.lax.broadcasted_iota(jnp.int32, (B_TILE, C_TILE), 1)
    is_tgt = cols == lbl[:, None]
    t_part = jnp.sum(jnp.where(is_tgt, logits, 0.0), axis=1)

    @pl.when(j == 0)
    def _():
        s_scr[i] = sum_e
        t_scr[i] = t_part

    @pl.when(j > 0)
    def _():
        s_scr[i] = s_scr[i] + sum_e
        t_scr[i] = t_scr[i] + t_part

    @pl.when(j == NC - 1)
    def _():
        t = t_scr[i]
        tm = t - MARGIN
        s = s_scr[i] - N_PAD - jnp.exp(t) + jnp.exp(tm)
        out_ref[0, 0, :] = jnp.log(s) - tm


def kernel(feature, label, centers):
    fn = _row_normalize(feature, 512)
    c_pad = jnp.pad(centers, ((0, C_PAD - NUM_CLASSES), (0, 0)))
    cn = _row_normalize(c_pad, 1024)
    lbl3 = label.reshape(NB, 1, B_TILE)

    out = pl.pallas_call(
        _body,
        grid=(NC, NB),
        in_specs=[
            pl.BlockSpec((B_TILE, DIM), lambda j, i: (i, 0)),
            pl.BlockSpec((C_TILE, DIM), lambda j, i: (j, 0)),
            pl.BlockSpec((1, 1, B_TILE), lambda j, i: (i, 0, 0)),
        ],
        out_specs=pl.BlockSpec((1, 1, B_TILE), lambda j, i: (i, 0, 0)),
        out_shape=jax.ShapeDtypeStruct((NB, 1, B_TILE), jnp.float32),
        scratch_shapes=[
            pltpu.VMEM((NB, B_TILE), jnp.float32),
            pltpu.VMEM((NB, B_TILE), jnp.float32),
        ],
    )(fn, cn, lbl3)
    return out.reshape(BATCH)


# speedup vs baseline: 2.0233x; 1.1047x over previous
import jax
import jax.numpy as jnp
from jax.experimental import pallas as pl
from jax.experimental.pallas import tpu as pltpu

BATCH = 4096
DIM = 512
NUM_CLASSES = 10000
MARGIN = 0.35

B_TILE = 512
C_TILE = 2048
C_PAD = 10240
NB = BATCH // B_TILE
NC = C_PAD // C_TILE
N_PAD = float(C_PAD - NUM_CLASSES)


def _norm_body(x_ref, o_ref):
    x = x_ref[...]
    o_ref[...] = x / (jnp.sqrt(jnp.sum(x * x, axis=1, keepdims=True)) + 1e-12)


def _row_normalize(x, row_tile):
    rows = x.shape[0]
    return pl.pallas_call(
        _norm_body,
        grid=(rows // row_tile,),
        in_specs=[pl.BlockSpec((row_tile, DIM), lambda i: (i, 0))],
        out_specs=pl.BlockSpec((row_tile, DIM), lambda i: (i, 0)),
        out_shape=jax.ShapeDtypeStruct(x.shape, jnp.float32),
    )(x)


LANES = 128
NCHUNK = C_TILE // LANES


def _body(f_ref, c_ref, lbl_ref, out_ref, s_scr, t_scr):
    j = pl.program_id(0)
    i = pl.program_id(1)

    logits = jax.lax.dot_general(
        f_ref[...], c_ref[...], (((1,), (1,)), ((), ())),
        preferred_element_type=jnp.float32,
    )

    e = jnp.exp(logits)
    lbl = lbl_ref[0, 0, :]
    cols = j * C_TILE + jax.lax.broadcasted_iota(jnp.int32, (B_TILE, C_TILE), 1)
    masked = jnp.where(cols == lbl[:, None], logits, 0.0)

    sum_e = e[:, :LANES]
    t_part = masked[:, :LANES]
    for k in range(1, NCHUNK):
        sum_e = sum_e + e[:, k * LANES:(k + 1) * LANES]
        t_part = t_part + masked[:, k * LANES:(k + 1) * LANES]

    @pl.when(j == 0)
    def _():
        s_scr[i] = sum_e
        t_scr[i] = t_part

    @pl.when(j > 0)
    def _():
        s_scr[i] = s_scr[i] + sum_e
        t_scr[i] = t_scr[i] + t_part

    @pl.when(j == NC - 1)
    def _():
        t = jnp.sum(t_scr[i], axis=1)
        tm = t - MARGIN
        s = jnp.sum(s_scr[i], axis=1) - N_PAD - jnp.exp(t) + jnp.exp(tm)
        out_ref[0, 0, :] = jnp.log(s) - tm


def kernel(feature, label, centers):
    fn = _row_normalize(feature, 512)
    c_pad = jnp.pad(centers, ((0, C_PAD - NUM_CLASSES), (0, 0)))
    cn = _row_normalize(c_pad, 1024)
    lbl3 = label.reshape(NB, 1, B_TILE)

    out = pl.pallas_call(
        _body,
        grid=(NC, NB),
        in_specs=[
            pl.BlockSpec((B_TILE, DIM), lambda j, i: (i, 0)),
            pl.BlockSpec((C_TILE, DIM), lambda j, i: (j, 0)),
            pl.BlockSpec((1, 1, B_TILE), lambda j, i: (i, 0, 0)),
        ],
        out_specs=pl.BlockSpec((1, 1, B_TILE), lambda j, i: (i, 0, 0)),
        out_shape=jax.ShapeDtypeStruct((NB, 1, B_TILE), jnp.float32),
        scratch_shapes=[
            pltpu.VMEM((NB, B_TILE, LANES), jnp.float32),
            pltpu.VMEM((NB, B_TILE, LANES), jnp.float32),
        ],
    )(fn, cn, lbl3)
    return out.reshape(BATCH)
